# baseline (device time: 14293 ns/iter reference)
import jax
import jax.numpy as jnp
from jax import lax
from jax.experimental import pallas as pl
from jax.experimental.pallas import tpu as pltpu

N_DEV = 4
N_TOK = 256
D_IN = 128
D_OUT = 256
E_PER = 2
CAP = 25


def kernel(x, router_W, route_idx, expert_W):
    del router_W

    def body(x_ref, idx_ref, ew_ref, out_ref, comm_ref, send_sems, recv_sems):
        my = lax.axis_index("i")
        left = lax.rem(my + N_DEV - 1, N_DEV)
        right = lax.rem(my + 1, N_DEV)

        route = idx_ref[:, :]
        e0 = my * E_PER
        ind0 = (route == e0).astype(jnp.float32)
        ind1 = (route == e0 + 1).astype(jnp.float32)
        ind = jnp.concatenate([ind0, ind1], axis=1)

        row = lax.broadcasted_iota(jnp.int32, (N_TOK, N_TOK), 0)
        col = lax.broadcasted_iota(jnp.int32, (N_TOK, N_TOK), 1)
        tri = (col <= row).astype(jnp.float32)
        cnt = jnp.dot(tri, ind, preferred_element_type=jnp.float32)
        keep0 = (ind0 > 0.0) & (cnt[:, 0:1] <= CAP)
        keep1 = (ind1 > 0.0) & (cnt[:, 1:2] <= CAP)

        xb = x_ref[:, :].astype(jnp.bfloat16)
        y0 = jnp.dot(xb, ew_ref[0, :, :].astype(jnp.bfloat16),
                     preferred_element_type=jnp.float32)
        y1 = jnp.dot(xb, ew_ref[1, :, :].astype(jnp.bfloat16),
                     preferred_element_type=jnp.float32)
        partial = jnp.where(keep0, y0, 0.0) + jnp.where(keep1, y1, 0.0)

        out_ref[:, :] = partial
        comm_ref[0, :, :] = partial.astype(jnp.bfloat16)

        barrier_sem = pltpu.get_barrier_semaphore()
        for nbr in (left, right):
            pl.semaphore_signal(barrier_sem, inc=1, device_id=(nbr,),
                                device_id_type=pl.DeviceIdType.MESH)
        pl.semaphore_wait(barrier_sem, 2)

        for h in range(N_DEV - 1):
            rdma = pltpu.make_async_remote_copy(
                src_ref=comm_ref.at[h],
                dst_ref=comm_ref.at[h + 1],
                send_sem=send_sems.at[h],
                recv_sem=recv_sems.at[h],
                device_id=(right,),
                device_id_type=pl.DeviceIdType.MESH,
            )
            rdma.start()
            rdma.wait()
            out_ref[:, :] = out_ref[:, :] + comm_ref[h + 1, :, :].astype(jnp.float32)

    out_shape = jax.ShapeDtypeStruct((N_TOK, D_OUT), jnp.float32)
    return pl.pallas_call(
        body,
        out_shape=out_shape,
        in_specs=[
            pl.BlockSpec(memory_space=pltpu.VMEM),
            pl.BlockSpec(memory_space=pltpu.VMEM),
            pl.BlockSpec(memory_space=pltpu.VMEM),
        ],
        out_specs=pl.BlockSpec(memory_space=pltpu.VMEM),
        scratch_shapes=[
            pltpu.VMEM((N_DEV, N_TOK, D_OUT), jnp.bfloat16),
            pltpu.SemaphoreType.DMA((N_DEV - 1,)),
            pltpu.SemaphoreType.DMA((N_DEV - 1,)),
        ],
        compiler_params=pltpu.CompilerParams(collective_id=0),
    )(x, route_idx, expert_W)


# device time: 10363 ns/iter; 1.3792x vs baseline; 1.3792x over previous
import jax
import jax.numpy as jnp
from jax import lax
from jax.experimental import pallas as pl
from jax.experimental.pallas import tpu as pltpu

N_DEV = 4
N_TOK = 256
D_IN = 128
D_OUT = 256
E_PER = 2
CAP = 25


def kernel(x, router_W, route_idx, expert_W):
    del router_W

    def body(x_ref, idx_ref, ew_ref, out_ref, comm_ref, send_sems, recv_sems):
        my = lax.axis_index("i")
        left = lax.rem(my + N_DEV - 1, N_DEV)
        right = lax.rem(my + 1, N_DEV)
        diag = lax.rem(my + 2, N_DEV)

        barrier_sem = pltpu.get_barrier_semaphore()
        for nbr in (left, right, diag):
            pl.semaphore_signal(barrier_sem, inc=1, device_id=(nbr,),
                                device_id_type=pl.DeviceIdType.MESH)

        route = idx_ref[:, :]
        e0 = my * E_PER
        ind0 = (route == e0).astype(jnp.float32)
        ind1 = (route == e0 + 1).astype(jnp.float32)
        ind = jnp.concatenate([ind0, ind1], axis=1)

        row = lax.broadcasted_iota(jnp.int32, (N_TOK, N_TOK), 0)
        col = lax.broadcasted_iota(jnp.int32, (N_TOK, N_TOK), 1)
        tri = (col <= row).astype(jnp.float32)
        cnt = jnp.dot(tri, ind, preferred_element_type=jnp.float32)
        keep0 = (ind0 > 0.0) & (cnt[:, 0:1] <= CAP)
        keep1 = (ind1 > 0.0) & (cnt[:, 1:2] <= CAP)

        xb = x_ref[:, :].astype(jnp.bfloat16)
        y0 = jnp.dot(xb, ew_ref[0, :, :].astype(jnp.bfloat16),
                     preferred_element_type=jnp.float32)
        y1 = jnp.dot(xb, ew_ref[1, :, :].astype(jnp.bfloat16),
                     preferred_element_type=jnp.float32)
        partial = jnp.where(keep0, y0, 0.0) + jnp.where(keep1, y1, 0.0)

        out_ref[:, :] = partial
        comm_ref[0, :, :] = partial.astype(jnp.bfloat16)

        pl.semaphore_wait(barrier_sem, N_DEV - 1)

        sends = []
        for slot, dest in ((3, diag), (2, left), (1, right)):
            rdma = pltpu.make_async_remote_copy(
                src_ref=comm_ref.at[0],
                dst_ref=comm_ref.at[slot],
                send_sem=send_sems.at[slot - 1],
                recv_sem=recv_sems.at[slot - 1],
                device_id=(dest,),
                device_id_type=pl.DeviceIdType.MESH,
            )
            rdma.start()
            sends.append(rdma)

        for slot in (1, 2, 3):
            recv = pltpu.make_async_remote_copy(
                src_ref=comm_ref.at[0],
                dst_ref=comm_ref.at[slot],
                send_sem=send_sems.at[slot - 1],
                recv_sem=recv_sems.at[slot - 1],
                device_id=(right,),
                device_id_type=pl.DeviceIdType.MESH,
            )
            recv.wait_recv()
            out_ref[:, :] = out_ref[:, :] + comm_ref[slot, :, :].astype(jnp.float32)

        for rdma in sends:
            rdma.wait_send()

    out_shape = jax.ShapeDtypeStruct((N_TOK, D_OUT), jnp.float32)
    return pl.pallas_call(
        body,
        out_shape=out_shape,
        in_specs=[
            pl.BlockSpec(memory_space=pltpu.VMEM),
            pl.BlockSpec(memory_space=pltpu.VMEM),
            pl.BlockSpec(memory_space=pltpu.VMEM),
        ],
        out_specs=pl.BlockSpec(memory_space=pltpu.VMEM),
        scratch_shapes=[
            pltpu.VMEM((N_DEV, N_TOK, D_OUT), jnp.bfloat16),
            pltpu.SemaphoreType.DMA((N_DEV - 1,)),
            pltpu.SemaphoreType.DMA((N_DEV - 1,)),
        ],
        compiler_params=pltpu.CompilerParams(collective_id=0),
    )(x, route_idx, expert_W)
